# baseline (speedup 1.0000x reference)
import functools

import jax
import jax.numpy as jnp
from jax import lax
from jax.experimental import pallas as pl
from jax.experimental.pallas import tpu as pltpu
from jax.experimental.pallas import tpu_sc as plsc

VOCAB = 100000
EMBED = 128
HIDDEN = 2048
OUT = 1024
B = 16384
L = 50

NC = 2
NS = 16
NW = NC * NS
BPW = B // NW
NV = EMBED // 16
UNR = 25
NBUF = 8


BLK = 32
NBLK2 = BPW // (2 * BLK)


NSLICE = 4
SB = B // NSLICE
SPW = SB // NW
SNBLK2 = SPW // (2 * BLK)


def _embed_pool_body(x_hbm, table_hbm, out_hbm, idx_v, rows_v, acc_v,
                     sem_g, sem_o):
    wid = lax.axis_index("s") * NC + lax.axis_index("c")
    base = wid * SPW

    pltpu.sync_copy(x_hbm.at[pl.ds(base, SPW)], idx_v)

    for j in range(NBUF):
        pltpu.make_async_copy(
            table_hbm.at[idx_v.at[j]], rows_v.at[j], sem_g).start()

    def blkbody(ib, carry):
        for k in range(2):
            blkbase = ib * (2 * BLK) + k * BLK

            @pl.when(ib > 0)
            def _(k=k):
                pltpu.make_async_copy(
                    acc_v.at[k], out_hbm.at[pl.ds(base, BLK)], sem_o).wait()

            def ibody(i2, c, k=k, blkbase=blkbase):
                for j in range(NBUF):
                    off = i2 * NBUF + j
                    bb = blkbase + off
                    pltpu.make_async_copy(
                        table_hbm.at[idx_v.at[0]], rows_v.at[j],
                        sem_g).wait()

                    def lbody(e, acc, j=j):
                        acc = list(acc)
                        for u in range(UNR):
                            for v in range(NV):
                                acc[v] = acc[v] + rows_v[j, e * UNR + u,
                                                         pl.ds(16 * v, 16)]
                        return acc

                    acc = lax.fori_loop(
                        0, L // UNR, lbody,
                        [jnp.zeros((16,), jnp.float32)] * NV)
                    for v in range(NV):
                        acc_v[k, off, pl.ds(16 * v, 16)] = acc[v]

                    @pl.when(bb + NBUF < SPW)
                    def _(j=j, bb=bb):
                        pltpu.make_async_copy(
                            table_hbm.at[idx_v.at[bb + NBUF]], rows_v.at[j],
                            sem_g).start()
                return c

            lax.fori_loop(0, BLK // NBUF, ibody, 0)

            pltpu.make_async_copy(
                acc_v.at[k], out_hbm.at[pl.ds(base + blkbase, BLK)],
                sem_o).start()
        return carry

    lax.fori_loop(0, SNBLK2, blkbody, 0)

    for k in range(2):
        pltpu.make_async_copy(
            acc_v.at[k], out_hbm.at[pl.ds(base, BLK)], sem_o).wait()


_embed_pool = functools.partial(
    pl.kernel,
    mesh=plsc.VectorSubcoreMesh(core_axis_name="c", subcore_axis_name="s"),
    out_type=jax.ShapeDtypeStruct((SB, EMBED), jnp.float32),
    scratch_types=[
        pltpu.VMEM((SPW, L), jnp.int32),
        pltpu.VMEM((NBUF, L, EMBED), jnp.float32),
        pltpu.VMEM((2, BLK, EMBED), jnp.float32),
        pltpu.SemaphoreType.DMA,
        pltpu.SemaphoreType.DMA,
    ],
)(_embed_pool_body)


BM = 512
NSTEP = SB // BM


def _mlp_kernel(x_ref, w1_ref, b1_ref, w2_ref, b2_ref, o_ref):
    x = x_ref[...] * (1.0 / L)
    h = jnp.dot(x, w1_ref[...], preferred_element_type=jnp.float32)
    h = jnp.maximum(h + b1_ref[...], 0.0)
    o = jnp.dot(h.astype(jnp.bfloat16), w2_ref[...],
                preferred_element_type=jnp.float32)
    o_ref[...] = o + b2_ref[...]


def _mlp_kernel_carry(carry_ref, x_ref, w1_ref, b1_ref, w2_ref, b2_ref,
                      o_ref):
    del carry_ref
    _mlp_kernel(x_ref, w1_ref, b1_ref, w2_ref, b2_ref, o_ref)


_WSPECS = [
    pl.BlockSpec((EMBED, HIDDEN), lambda i: (0, 0)),
    pl.BlockSpec((1, HIDDEN), lambda i: (0, 0)),
    pl.BlockSpec((HIDDEN, OUT), lambda i: (0, 0)),
    pl.BlockSpec((1, OUT), lambda i: (0, 0)),
]


def _mlp_slice(carry, pooled_sum, W1, b1, W2, b2, s):
    out_spec = pl.BlockSpec((BM, OUT), lambda i, s=s: (s * NSTEP + i, 0))
    out_shape = jax.ShapeDtypeStruct((B, OUT), jnp.float32)
    x_spec = pl.BlockSpec((BM, EMBED), lambda i: (i, 0))
    args = (pooled_sum, W1, b1.reshape(1, HIDDEN),
            W2.astype(jnp.bfloat16), b2.reshape(1, OUT))
    if carry is None:
        return pl.pallas_call(
            _mlp_kernel,
            grid=(NSTEP,),
            in_specs=[x_spec] + _WSPECS,
            out_specs=out_spec,
            out_shape=out_shape,
        )(*args)
    return pl.pallas_call(
        _mlp_kernel_carry,
        grid=(NSTEP,),
        in_specs=[pl.BlockSpec(memory_space=pl.ANY), x_spec] + _WSPECS,
        out_specs=out_spec,
        out_shape=out_shape,
        input_output_aliases={0: 0},
    )(carry, *args)


def kernel(x, table, W1, b1, W2, b2):
    pooled = [
        _embed_pool(lax.slice_in_dim(x, s * SB, (s + 1) * SB), table)
        for s in range(NSLICE)
    ]
    out = None
    for s in range(NSLICE):
        out = _mlp_slice(out, pooled[s], W1, b1, W2, b2, s)
    return out


# speedup vs baseline: 1.6899x; 1.6899x over previous
import functools

import jax
import jax.numpy as jnp
from jax import lax
from jax.experimental import pallas as pl
from jax.experimental.pallas import tpu as pltpu
from jax.experimental.pallas import tpu_sc as plsc

VOCAB = 100000
EMBED = 128
HIDDEN = 2048
OUT = 1024
B = 16384
L = 50

NC = 2
NS = 16
NW = NC * NS
BPW = B // NW
NV = EMBED // 16
UNR = 10
NBUF = 8


BLK = 32
NBLK2 = BPW // (2 * BLK)


NSLICE = 2
SB = B // NSLICE
SPW = SB // NW
SNBLK2 = SPW // (2 * BLK)


def _embed_pool_body(x_hbm, table_hbm, out_hbm, idx_v, rows_v, acc_v,
                     sem_g, sem_o):
    wid = lax.axis_index("s") * NC + lax.axis_index("c")
    base = wid * SPW

    pltpu.sync_copy(x_hbm.at[pl.ds(base, SPW)], idx_v)

    for j in range(NBUF):
        pltpu.make_async_copy(
            table_hbm.at[idx_v.at[j]], rows_v.at[j], sem_g).start()

    def blkbody(ib, carry):
        for k in range(2):
            blkbase = ib * (2 * BLK) + k * BLK

            @pl.when(ib > 0)
            def _(k=k):
                pltpu.make_async_copy(
                    acc_v.at[k], out_hbm.at[pl.ds(base, BLK)], sem_o).wait()

            def ibody(i2, c, k=k, blkbase=blkbase):
                for j in range(NBUF):
                    off = i2 * NBUF + j
                    bb = blkbase + off
                    pltpu.make_async_copy(
                        table_hbm.at[idx_v.at[0]], rows_v.at[j],
                        sem_g).wait()

                    def lbody(e, acc, j=j):
                        acc = list(acc)
                        for u in range(UNR):
                            for v in range(NV):
                                acc[v] = acc[v] + rows_v[j, e * UNR + u,
                                                         pl.ds(16 * v, 16)]
                        return acc

                    acc = lax.fori_loop(
                        0, L // UNR, lbody,
                        [jnp.zeros((16,), jnp.float32)] * NV)
                    for v in range(NV):
                        acc_v[k, off, pl.ds(16 * v, 16)] = acc[v]

                    @pl.when(bb + NBUF < SPW)
                    def _(j=j, bb=bb):
                        pltpu.make_async_copy(
                            table_hbm.at[idx_v.at[bb + NBUF]], rows_v.at[j],
                            sem_g).start()
                return c

            lax.fori_loop(0, BLK // NBUF, ibody, 0)

            pltpu.make_async_copy(
                acc_v.at[k], out_hbm.at[pl.ds(base + blkbase, BLK)],
                sem_o).start()
        return carry

    lax.fori_loop(0, SNBLK2, blkbody, 0)

    for k in range(2):
        pltpu.make_async_copy(
            acc_v.at[k], out_hbm.at[pl.ds(base, BLK)], sem_o).wait()


_embed_pool = functools.partial(
    pl.kernel,
    mesh=plsc.VectorSubcoreMesh(core_axis_name="c", subcore_axis_name="s"),
    out_type=jax.ShapeDtypeStruct((SB, EMBED), jnp.float32),
    scratch_types=[
        pltpu.VMEM((SPW, L), jnp.int32),
        pltpu.VMEM((NBUF, L, EMBED), jnp.float32),
        pltpu.VMEM((2, BLK, EMBED), jnp.float32),
        pltpu.SemaphoreType.DMA,
        pltpu.SemaphoreType.DMA,
    ],
)(_embed_pool_body)


BM = 512
NSTEP = SB // BM


def _mlp_kernel(x_ref, w1_ref, b1_ref, w2_ref, b2_ref, o_ref):
    x = x_ref[...] * (1.0 / L)
    h = jnp.dot(x, w1_ref[...], preferred_element_type=jnp.float32)
    h = jnp.maximum(h + b1_ref[...], 0.0)
    o = jnp.dot(h.astype(jnp.bfloat16), w2_ref[...],
                preferred_element_type=jnp.float32)
    o_ref[...] = o + b2_ref[...]


def _mlp_kernel_carry(carry_ref, x_ref, w1_ref, b1_ref, w2_ref, b2_ref,
                      o_ref):
    del carry_ref
    _mlp_kernel(x_ref, w1_ref, b1_ref, w2_ref, b2_ref, o_ref)


_WSPECS = [
    pl.BlockSpec((EMBED, HIDDEN), lambda i: (0, 0)),
    pl.BlockSpec((1, HIDDEN), lambda i: (0, 0)),
    pl.BlockSpec((HIDDEN, OUT), lambda i: (0, 0)),
    pl.BlockSpec((1, OUT), lambda i: (0, 0)),
]


def _mlp_slice(carry, pooled_sum, W1, b1, W2, b2, s):
    out_spec = pl.BlockSpec((BM, OUT), lambda i, s=s: (s * NSTEP + i, 0))
    out_shape = jax.ShapeDtypeStruct((B, OUT), jnp.float32)
    x_spec = pl.BlockSpec((BM, EMBED), lambda i: (i, 0))
    args = (pooled_sum, W1, b1.reshape(1, HIDDEN),
            W2.astype(jnp.bfloat16), b2.reshape(1, OUT))
    if carry is None:
        return pl.pallas_call(
            _mlp_kernel,
            grid=(NSTEP,),
            in_specs=[x_spec] + _WSPECS,
            out_specs=out_spec,
            out_shape=out_shape,
        )(*args)
    return pl.pallas_call(
        _mlp_kernel_carry,
        grid=(NSTEP,),
        in_specs=[pl.BlockSpec(memory_space=pl.ANY), x_spec] + _WSPECS,
        out_specs=out_spec,
        out_shape=out_shape,
        input_output_aliases={0: 0},
    )(carry, *args)


def kernel(x, table, W1, b1, W2, b2):
    pooled = [
        _embed_pool(lax.slice_in_dim(x, s * SB, (s + 1) * SB), table)
        for s in range(NSLICE)
    ]
    out = None
    for s in range(NSLICE):
        out = _mlp_slice(out, pooled[s], W1, b1, W2, b2, s)
    return out


# speedup vs baseline: 1.7023x; 1.0073x over previous
import functools

import jax
import jax.numpy as jnp
from jax import lax
from jax.experimental import pallas as pl
from jax.experimental.pallas import tpu as pltpu
from jax.experimental.pallas import tpu_sc as plsc

VOCAB = 100000
EMBED = 128
HIDDEN = 2048
OUT = 1024
B = 16384
L = 50

NC = 2
NS = 16
NW = NC * NS
BPW = B // NW
NV = EMBED // 16
UNR = 10
NBUF = 8


BLK = 32
NBLK2 = BPW // (2 * BLK)


NSLICE = 2
SB = B // NSLICE
SPW = SB // NW
SNBLK2 = SPW // (2 * BLK)


def _embed_pool_body(x_hbm, table_hbm, out_hbm, idx_v, rows_v, acc_v,
                     sem_g, sem_o, *, slice_base):
    wid = lax.axis_index("s") * NC + lax.axis_index("c")
    base = slice_base + wid * SPW
    obase = wid * SPW

    pltpu.sync_copy(x_hbm.at[pl.ds(base, SPW)], idx_v)

    for j in range(NBUF):
        pltpu.make_async_copy(
            table_hbm.at[idx_v.at[j]], rows_v.at[j], sem_g).start()

    def blkbody(ib, carry):
        for k in range(2):
            blkbase = ib * (2 * BLK) + k * BLK

            @pl.when(ib > 0)
            def _(k=k):
                pltpu.make_async_copy(
                    acc_v.at[k], out_hbm.at[pl.ds(obase, BLK)], sem_o).wait()

            def ibody(i2, c, k=k, blkbase=blkbase):
                for j in range(NBUF):
                    off = i2 * NBUF + j
                    bb = blkbase + off
                    pltpu.make_async_copy(
                        table_hbm.at[idx_v.at[0]], rows_v.at[j],
                        sem_g).wait()

                    def lbody(e, acc, j=j):
                        acc = list(acc)
                        for u in range(UNR):
                            for v in range(NV):
                                acc[v] = acc[v] + rows_v[j, e * UNR + u,
                                                         pl.ds(16 * v, 16)]
                        return acc

                    acc = lax.fori_loop(
                        0, L // UNR, lbody,
                        [jnp.zeros((16,), jnp.float32)] * NV)
                    for v in range(NV):
                        acc_v[k, off, pl.ds(16 * v, 16)] = acc[v]

                    @pl.when(bb + NBUF < SPW)
                    def _(j=j, bb=bb):
                        pltpu.make_async_copy(
                            table_hbm.at[idx_v.at[bb + NBUF]], rows_v.at[j],
                            sem_g).start()
                return c

            lax.fori_loop(0, BLK // NBUF, ibody, 0)

            pltpu.make_async_copy(
                acc_v.at[k], out_hbm.at[pl.ds(obase + blkbase, BLK)],
                sem_o).start()
        return carry

    lax.fori_loop(0, SNBLK2, blkbody, 0)

    for k in range(2):
        pltpu.make_async_copy(
            acc_v.at[k], out_hbm.at[pl.ds(obase, BLK)], sem_o).wait()


def _make_embed_pool(slice_base):
    return functools.partial(
        pl.kernel,
        mesh=plsc.VectorSubcoreMesh(core_axis_name="c", subcore_axis_name="s"),
        out_type=jax.ShapeDtypeStruct((SB, EMBED), jnp.float32),
        scratch_types=[
            pltpu.VMEM((SPW, L), jnp.int32),
            pltpu.VMEM((NBUF, L, EMBED), jnp.float32),
            pltpu.VMEM((2, BLK, EMBED), jnp.float32),
            pltpu.SemaphoreType.DMA,
            pltpu.SemaphoreType.DMA,
        ],
    )(functools.partial(_embed_pool_body, slice_base=slice_base))


_embed_pools = [_make_embed_pool(s * SB) for s in range(NSLICE)]


BM = 1024
NSTEP = SB // BM


def _mlp_kernel(x_ref, w1_ref, b1_ref, w2_ref, b2_ref, o_ref):
    x = x_ref[...] * (1.0 / L)
    h = jnp.dot(x, w1_ref[...], preferred_element_type=jnp.float32)
    h = jnp.maximum(h + b1_ref[...], 0.0)
    o = jnp.dot(h.astype(jnp.bfloat16), w2_ref[...],
                preferred_element_type=jnp.float32)
    o_ref[...] = o + b2_ref[...]


def _mlp_kernel_carry(carry_ref, x_ref, w1_ref, b1_ref, w2_ref, b2_ref,
                      o_ref):
    del carry_ref
    _mlp_kernel(x_ref, w1_ref, b1_ref, w2_ref, b2_ref, o_ref)


_WSPECS = [
    pl.BlockSpec((EMBED, HIDDEN), lambda i: (0, 0)),
    pl.BlockSpec((1, HIDDEN), lambda i: (0, 0)),
    pl.BlockSpec((HIDDEN, OUT), lambda i: (0, 0)),
    pl.BlockSpec((1, OUT), lambda i: (0, 0)),
]


def _mlp_slice(carry, pooled_sum, W1, b1, W2, b2, s):
    out_spec = pl.BlockSpec((BM, OUT), lambda i, s=s: (s * NSTEP + i, 0))
    out_shape = jax.ShapeDtypeStruct((B, OUT), jnp.float32)
    x_spec = pl.BlockSpec((BM, EMBED), lambda i: (i, 0))
    args = (pooled_sum, W1, b1.reshape(1, HIDDEN),
            W2.astype(jnp.bfloat16), b2.reshape(1, OUT))
    if carry is None:
        return pl.pallas_call(
            _mlp_kernel,
            grid=(NSTEP,),
            in_specs=[x_spec] + _WSPECS,
            out_specs=out_spec,
            out_shape=out_shape,
        )(*args)
    return pl.pallas_call(
        _mlp_kernel_carry,
        grid=(NSTEP,),
        in_specs=[pl.BlockSpec(memory_space=pl.ANY), x_spec] + _WSPECS,
        out_specs=out_spec,
        out_shape=out_shape,
        input_output_aliases={0: 0},
    )(carry, *args)


def kernel(x, table, W1, b1, W2, b2):
    pooled = [_embed_pools[s](x, table) for s in range(NSLICE)]
    out = None
    for s in range(NSLICE):
        out = _mlp_slice(out, pooled[s], W1, b1, W2, b2, s)
    return out


# speedup vs baseline: 1.7923x; 1.0529x over previous
import functools

import jax
import jax.numpy as jnp
from jax import lax
from jax.experimental import pallas as pl
from jax.experimental.pallas import tpu as pltpu
from jax.experimental.pallas import tpu_sc as plsc

VOCAB = 100000
EMBED = 128
HIDDEN = 2048
OUT = 1024
B = 16384
L = 50

NC = 2
NS = 16
NW = NC * NS
BPW = B // NW
NV = EMBED // 16
UNR = 10
NBUF = 8


BLK = 32


SLICES = (8192, 6144, 2048)


def _embed_pool_body(x_hbm, table_hbm, out_hbm, idx_v, rows_v, acc_v,
                     sem_g, sem_o, *, slice_base, spw):
    SPW = spw
    SNBLK2 = SPW // (2 * BLK)
    wid = lax.axis_index("s") * NC + lax.axis_index("c")
    base = slice_base + wid * SPW
    obase = wid * SPW

    pltpu.sync_copy(x_hbm.at[pl.ds(base, SPW)], idx_v)

    for j in range(NBUF):
        pltpu.make_async_copy(
            table_hbm.at[idx_v.at[j]], rows_v.at[pl.ds(j * L, L)],
            sem_g).start()

    def blkbody(ib, carry):
        for k in range(2):
            blkbase = ib * (2 * BLK) + k * BLK

            @pl.when(ib > 0)
            def _(k=k):
                pltpu.make_async_copy(
                    acc_v.at[k], out_hbm.at[pl.ds(obase, BLK)], sem_o).wait()

            def ibody(i2, c, k=k, blkbase=blkbase):
                for j in range(NBUF):
                    off = i2 * NBUF + j
                    bb = blkbase + off
                    pltpu.make_async_copy(
                        table_hbm.at[idx_v.at[0]],
                        rows_v.at[pl.ds(j * L, L)], sem_g).wait()

                    def lbody(e, acc, j=j):
                        acc = list(acc)
                        for u in range(UNR):
                            for v in range(NV):
                                acc[v] = acc[v] + rows_v[j * L + e * UNR + u,
                                                         pl.ds(16 * v, 16)]
                        return acc

                    acc = lax.fori_loop(
                        0, L // UNR, lbody,
                        [jnp.zeros((16,), jnp.float32)] * NV)
                    for v in range(NV):
                        acc_v[k, off, pl.ds(16 * v, 16)] = acc[v]

                    @pl.when(bb + NBUF < SPW)
                    def _(j=j, bb=bb):
                        pltpu.make_async_copy(
                            table_hbm.at[idx_v.at[bb + NBUF]],
                            rows_v.at[pl.ds(j * L, L)], sem_g).start()
                return c

            lax.fori_loop(0, BLK // NBUF, ibody, 0)

            pltpu.make_async_copy(
                acc_v.at[k], out_hbm.at[pl.ds(obase + blkbase, BLK)],
                sem_o).start()
        return carry

    lax.fori_loop(0, SNBLK2, blkbody, 0)

    for k in range(2):
        pltpu.make_async_copy(
            acc_v.at[k], out_hbm.at[pl.ds(obase, BLK)], sem_o).wait()


def _make_embed_pool(slice_base, sb):
    spw = sb // NW
    return functools.partial(
        pl.kernel,
        mesh=plsc.VectorSubcoreMesh(core_axis_name="c", subcore_axis_name="s"),
        out_type=jax.ShapeDtypeStruct((sb, EMBED), jnp.float32),
        scratch_types=[
            pltpu.VMEM((spw, L), jnp.int32),
            pltpu.VMEM((NBUF * L, EMBED), jnp.float32),
            pltpu.VMEM((2, BLK, EMBED), jnp.float32),
            pltpu.SemaphoreType.DMA,
            pltpu.SemaphoreType.DMA,
        ],
    )(functools.partial(_embed_pool_body, slice_base=slice_base, spw=spw))


_SLICE_BASES = tuple(sum(SLICES[:s]) for s in range(len(SLICES)))
_embed_pools = [_make_embed_pool(_SLICE_BASES[s], SLICES[s])
                for s in range(len(SLICES))]


BM = 1024


def _mlp_kernel(x_ref, w1_ref, b1_ref, w2_ref, b2_ref, o_ref):
    x = x_ref[...] * (1.0 / L)
    h = jnp.dot(x, w1_ref[...], preferred_element_type=jnp.float32)
    h = jnp.maximum(h + b1_ref[...], 0.0)
    o = jnp.dot(h.astype(jnp.bfloat16), w2_ref[...],
                preferred_element_type=jnp.float32)
    o_ref[...] = o + b2_ref[...]


def _mlp_kernel_carry(carry_ref, x_ref, w1_ref, b1_ref, w2_ref, b2_ref,
                      o_ref):
    del carry_ref
    _mlp_kernel(x_ref, w1_ref, b1_ref, w2_ref, b2_ref, o_ref)


_WSPECS = [
    pl.BlockSpec((EMBED, HIDDEN), lambda i: (0, 0)),
    pl.BlockSpec((1, HIDDEN), lambda i: (0, 0)),
    pl.BlockSpec((HIDDEN, OUT), lambda i: (0, 0)),
    pl.BlockSpec((1, OUT), lambda i: (0, 0)),
]


def _mlp_slice(carry, pooled_sum, W1, b1, W2, b2, s):
    base, size = _SLICE_BASES[s], SLICES[s]
    nstep = size // BM
    out_spec = pl.BlockSpec((BM, OUT), lambda i, base=base: (base // BM + i, 0))
    out_shape = jax.ShapeDtypeStruct((B, OUT), jnp.float32)
    x_spec = pl.BlockSpec((BM, EMBED), lambda i: (i, 0))
    args = (pooled_sum, W1, b1.reshape(1, HIDDEN),
            W2.astype(jnp.bfloat16), b2.reshape(1, OUT))
    if carry is None:
        return pl.pallas_call(
            _mlp_kernel,
            grid=(nstep,),
            in_specs=[x_spec] + _WSPECS,
            out_specs=out_spec,
            out_shape=out_shape,
        )(*args)
    return pl.pallas_call(
        _mlp_kernel_carry,
        grid=(nstep,),
        in_specs=[pl.BlockSpec(memory_space=pl.ANY), x_spec] + _WSPECS,
        out_specs=out_spec,
        out_shape=out_shape,
        input_output_aliases={0: 0},
    )(carry, *args)


def kernel(x, table, W1, b1, W2, b2):
    pooled = [_embed_pools[s](x, table) for s in range(len(SLICES))]
    out = None
    for s in range(len(SLICES)):
        out = _mlp_slice(out, pooled[s], W1, b1, W2, b2, s)
    return out
